# baseline (speedup 1.0000x reference)
import jax
import jax.numpy as jnp
from jax import lax
from jax.experimental import pallas as pl
from jax.experimental.pallas import tpu as pltpu
from jax.experimental.pallas import tpu_sc as plsc

_H = 224
_W = 224
_C = 192
_CHALF = 96
_B = 8
_NPLANES = _B * _C
_NC = 2
_NS = 16
_NW = _NC * _NS
_PPW = _NPLANES // _NW
_LANES = 16
_WV = _W // _LANES
_HH = _H // 2
_NCHUNK = 2 * _PPW

_GATHER_DN = lax.GatherDimensionNumbers(
    offset_dims=(), collapsed_slice_dims=(0,), start_index_map=(0,))


def _sc_body(x_hbm, emb_hbm, out_hbm, emb_v,
             buf0, buf1, buf2, gs0, gs1, gs2, ss0, ss1, ss2):
    cid = lax.axis_index("c")
    sid = lax.axis_index("s")
    wid = sid * _NC + cid
    p0 = wid * _PPW
    c0 = lax.rem(p0, _C)
    row_base = p0 * _H
    pltpu.sync_copy(emb_hbm.at[pl.ds(c0, _PPW)], emb_v)

    bufs = (buf0, buf1, buf2)
    gsems = (gs0, gs1, gs2)
    ssems = (ss0, ss1, ss2)

    def chunk_of(g):
        return pl.ds(row_base + g * _HH, _HH)

    def wait_gather(g, e):
        pltpu.make_async_copy(x_hbm.at[chunk_of(g)], bufs[e], gsems[e]).wait()

    def wait_scatter(g, e):
        pltpu.make_async_copy(
            bufs[e], out_hbm.at[chunk_of(g)], ssems[e]).wait()

    def add_pos_local(buf, g):
        plane_i = g // 2
        half = g - 2 * plane_i
        is_row = c0 + plane_i < _CHALF

        def row_channel(_):
            @plsc.parallel_loop(0, _HH // _LANES)
            def hb_body(hb):
                lr0 = hb * _LANES
                eh16 = emb_v[plane_i, pl.ds(half * _HH + lr0, _LANES)]
                for k in range(_LANES):
                    bcast = lax.gather(
                        eh16, jnp.full((_LANES, 1), k, jnp.int32),
                        _GATHER_DN, slice_sizes=(1,),
                        mode=lax.GatherScatterMode.PROMISE_IN_BOUNDS)
                    for j in range(_WV):
                        sl = pl.ds(j * _LANES, _LANES)
                        buf[lr0 + k, sl] = buf[lr0 + k, sl] + bcast

            return 0

        def col_channel(_):
            evs = [emb_v[plane_i, pl.ds(j * _LANES, _LANES)]
                   for j in range(_WV)]

            @plsc.parallel_loop(0, _HH, unroll=2)
            def r_body(r):
                for j in range(_WV):
                    sl = pl.ds(j * _LANES, _LANES)
                    buf[r, sl] = buf[r, sl] + evs[j]

            return 0

        lax.cond(is_row, row_channel, col_channel, 0)

    def do_chunk(g, e, first=False, tail=False):
        wait_gather(g, e)
        e2 = (e + 2) % 3
        if not first:
            wait_scatter(g - 1, e2)
        if not tail:
            pltpu.async_copy(x_hbm.at[chunk_of(g + 2)], bufs[e2], gsems[e2])
        add_pos_local(bufs[e], g)
        pltpu.async_copy(bufs[e], out_hbm.at[chunk_of(g)], ssems[e])

    pltpu.async_copy(x_hbm.at[chunk_of(0)], bufs[0], gsems[0])
    pltpu.async_copy(x_hbm.at[chunk_of(1)], bufs[1], gsems[1])

    do_chunk(0, 0, first=True)

    def t_body(t, carry):
        g = 3 * t + 1
        do_chunk(g, 1)
        do_chunk(g + 1, 2)
        do_chunk(g + 2, 0)
        return carry

    lax.fori_loop(0, (_NCHUNK - 3) // 3, t_body, 0, unroll=False)

    do_chunk(_NCHUNK - 2, 1, tail=True)
    do_chunk(_NCHUNK - 1, 2, tail=True)
    wait_scatter(_NCHUNK - 1, 2)


def kernel(x, row_embed, col_embed):
    b, c, h, w = x.shape
    emb = jnp.concatenate([row_embed.T, col_embed.T], axis=0)
    x2 = x.reshape(b * c * h, w)

    mesh = plsc.VectorSubcoreMesh(core_axis_name="c", subcore_axis_name="s")
    run = pl.kernel(
        _sc_body,
        mesh=mesh,
        out_type=jax.ShapeDtypeStruct((b * c * h, w), jnp.float32),
        scratch_types=[
            pltpu.VMEM((_PPW, _W), jnp.float32),
            pltpu.VMEM((_HH, _W), jnp.float32),
            pltpu.VMEM((_HH, _W), jnp.float32),
            pltpu.VMEM((_HH, _W), jnp.float32),
            pltpu.SemaphoreType.DMA,
            pltpu.SemaphoreType.DMA,
            pltpu.SemaphoreType.DMA,
            pltpu.SemaphoreType.DMA,
            pltpu.SemaphoreType.DMA,
            pltpu.SemaphoreType.DMA,
        ],
    )
    out2 = run(x2, emb)
    return out2.reshape(b, c, h, w)


# speedup vs baseline: 1.0104x; 1.0104x over previous
import jax
import jax.numpy as jnp
from jax import lax
from jax.experimental import pallas as pl
from jax.experimental.pallas import tpu as pltpu
from jax.experimental.pallas import tpu_sc as plsc

_H = 224
_W = 224
_C = 192
_CHALF = 96
_B = 8
_NPLANES = _B * _C
_NC = 2
_NS = 16
_NW = _NC * _NS
_PPW = _NPLANES // _NW
_LANES = 16
_WV = _W // _LANES
_HH = _H // 2
_NCHUNK = 2 * _PPW

_GATHER_DN = lax.GatherDimensionNumbers(
    offset_dims=(), collapsed_slice_dims=(0,), start_index_map=(0,))


def _sc_body(x_hbm, emb_hbm, out_hbm, emb_v,
             buf0, buf1, buf2, gs0, gs1, gs2, ss0, ss1, ss2):
    cid = lax.axis_index("c")
    sid = lax.axis_index("s")
    wid = sid * _NC + cid
    p0 = wid * _PPW
    c0 = lax.rem(p0, _C)
    row_base = p0 * _H
    pltpu.sync_copy(emb_hbm.at[pl.ds(c0, _PPW)], emb_v)

    bufs = (buf0, buf1, buf2)
    gsems = (gs0, gs1, gs2)
    ssems = (ss0, ss1, ss2)

    def chunk_of(g):
        return pl.ds(row_base + g * _HH, _HH)

    def wait_gather(g, e):
        pltpu.make_async_copy(x_hbm.at[chunk_of(g)], bufs[e], gsems[e]).wait()

    def wait_scatter(g, e):
        pltpu.make_async_copy(
            bufs[e], out_hbm.at[chunk_of(g)], ssems[e]).wait()

    def add_pos_local(buf, g):
        plane_i = g // 2
        half = g - 2 * plane_i
        is_row = c0 + plane_i < _CHALF

        def row_channel(_):
            @plsc.parallel_loop(0, _HH // _LANES)
            def hb_body(hb):
                lr0 = hb * _LANES
                eh16 = emb_v[plane_i, pl.ds(half * _HH + lr0, _LANES)]
                for k in range(_LANES):
                    bcast = lax.gather(
                        eh16, jnp.full((_LANES, 1), k, jnp.int32),
                        _GATHER_DN, slice_sizes=(1,),
                        mode=lax.GatherScatterMode.PROMISE_IN_BOUNDS)
                    for j in range(_WV):
                        sl = pl.ds(j * _LANES, _LANES)
                        buf[lr0 + k, sl] = buf[lr0 + k, sl] + bcast

            return 0

        def col_channel(_):
            evs = [emb_v[plane_i, pl.ds(j * _LANES, _LANES)]
                   for j in range(_WV)]

            @plsc.parallel_loop(0, _HH, unroll=2)
            def r_body(r):
                for j in range(_WV):
                    sl = pl.ds(j * _LANES, _LANES)
                    buf[r, sl] = buf[r, sl] + evs[j]

            return 0

        lax.cond(is_row, row_channel, col_channel, 0)

    def do_chunk(g, e, first=False, tail=False):
        wait_gather(g, e)
        add_pos_local(bufs[e], g)
        pltpu.async_copy(bufs[e], out_hbm.at[chunk_of(g)], ssems[e])
        e2 = (e + 2) % 3
        if not first:
            wait_scatter(g - 1, e2)
        if not tail:
            pltpu.async_copy(x_hbm.at[chunk_of(g + 2)], bufs[e2], gsems[e2])

    pltpu.async_copy(x_hbm.at[chunk_of(0)], bufs[0], gsems[0])
    pltpu.async_copy(x_hbm.at[chunk_of(1)], bufs[1], gsems[1])

    do_chunk(0, 0, first=True)

    def t_body(t, carry):
        g = 3 * t + 1
        do_chunk(g, 1)
        do_chunk(g + 1, 2)
        do_chunk(g + 2, 0)
        return carry

    lax.fori_loop(0, (_NCHUNK - 3) // 3, t_body, 0, unroll=False)

    do_chunk(_NCHUNK - 2, 1, tail=True)
    do_chunk(_NCHUNK - 1, 2, tail=True)
    wait_scatter(_NCHUNK - 1, 2)


def kernel(x, row_embed, col_embed):
    b, c, h, w = x.shape
    emb = jnp.concatenate([row_embed.T, col_embed.T], axis=0)
    x2 = x.reshape(b * c * h, w)

    mesh = plsc.VectorSubcoreMesh(core_axis_name="c", subcore_axis_name="s")
    run = pl.kernel(
        _sc_body,
        mesh=mesh,
        out_type=jax.ShapeDtypeStruct((b * c * h, w), jnp.float32),
        scratch_types=[
            pltpu.VMEM((_PPW, _W), jnp.float32),
            pltpu.VMEM((_HH, _W), jnp.float32),
            pltpu.VMEM((_HH, _W), jnp.float32),
            pltpu.VMEM((_HH, _W), jnp.float32),
            pltpu.SemaphoreType.DMA,
            pltpu.SemaphoreType.DMA,
            pltpu.SemaphoreType.DMA,
            pltpu.SemaphoreType.DMA,
            pltpu.SemaphoreType.DMA,
            pltpu.SemaphoreType.DMA,
        ],
    )
    out2 = run(x2, emb)
    return out2.reshape(b, c, h, w)


# speedup vs baseline: 1.0504x; 1.0396x over previous
import jax
import jax.numpy as jnp
from jax import lax
from jax.experimental import pallas as pl
from jax.experimental.pallas import tpu as pltpu
from jax.experimental.pallas import tpu_sc as plsc

_H = 224
_W = 224
_C = 192
_CHALF = 96
_B = 8
_NPLANES = _B * _C
_NC = 2
_NS = 16
_NW = _NC * _NS
_PPW = _NPLANES // _NW
_LANES = 16
_WV = _W // _LANES
_HH = _H // 2
_HQ = _HH // 2
_NCHUNK = 2 * _PPW

_GATHER_DN = lax.GatherDimensionNumbers(
    offset_dims=(), collapsed_slice_dims=(0,), start_index_map=(0,))


def _sc_body(x_hbm, emb_hbm, out_hbm, emb_v,
             buf0, buf1, buf2, spm, gs0, gs1, gs2, ss0, ss1):
    cid = lax.axis_index("c")
    sid = lax.axis_index("s")
    wid = sid * _NC + cid
    p0 = wid * _PPW
    c0 = lax.rem(p0, _C)
    row_base = p0 * _H
    pltpu.sync_copy(emb_hbm.at[pl.ds(c0, _PPW)], emb_v)

    bufs = (buf0, buf1, buf2)
    gsems = (gs0, gs1, gs2)
    ssems = (ss0, ss1)

    def chunk_of(g):
        return pl.ds(row_base + g * _HH, _HH)

    def wait_gather(g, e):
        pltpu.make_async_copy(x_hbm.at[chunk_of(g)], bufs[e], gsems[e]).wait()

    def half_of(g, u):
        return pl.ds(row_base + g * _HH + u * _HQ, _HQ)

    def wait_scatter(g, u):
        pltpu.make_async_copy(
            spm.at[sid, u], out_hbm.at[half_of(g, u)], ssems[u]).wait()

    def add_pos_local(buf, g):
        plane_i = g // 2
        half = g - 2 * plane_i
        is_row = c0 + plane_i < _CHALF

        def row_channel(_):
            @plsc.parallel_loop(0, _HH // _LANES)
            def hb_body(hb):
                lr0 = hb * _LANES
                eh16 = emb_v[plane_i, pl.ds(half * _HH + lr0, _LANES)]
                for k in range(_LANES):
                    bcast = lax.gather(
                        eh16, jnp.full((_LANES, 1), k, jnp.int32),
                        _GATHER_DN, slice_sizes=(1,),
                        mode=lax.GatherScatterMode.PROMISE_IN_BOUNDS)
                    for j in range(_WV):
                        sl = pl.ds(j * _LANES, _LANES)
                        buf[lr0 + k, sl] = buf[lr0 + k, sl] + bcast

            return 0

        def col_channel(_):
            evs = [emb_v[plane_i, pl.ds(j * _LANES, _LANES)]
                   for j in range(_WV)]

            @plsc.parallel_loop(0, _HH, unroll=2)
            def r_body(r):
                for j in range(_WV):
                    sl = pl.ds(j * _LANES, _LANES)
                    buf[r, sl] = buf[r, sl] + evs[j]

            return 0

        lax.cond(is_row, row_channel, col_channel, 0)

    def do_chunk(g, e, tail=False):
        wait_gather(g, e)
        if not tail:
            e2 = (e + 2) % 3
            pltpu.async_copy(x_hbm.at[chunk_of(g + 2)], bufs[e2], gsems[e2])
        add_pos_local(bufs[e], g)
        for u in range(2):
            wait_scatter(g - 1, u)
            pltpu.sync_copy(bufs[e].at[pl.ds(u * _HQ, _HQ), :], spm.at[sid, u])
            pltpu.async_copy(
                spm.at[sid, u], out_hbm.at[half_of(g, u)], ssems[u])

    pltpu.async_copy(x_hbm.at[chunk_of(0)], bufs[0], gsems[0])
    pltpu.async_copy(x_hbm.at[chunk_of(1)], bufs[1], gsems[1])
    for u in range(2):
        pltpu.async_copy(spm.at[sid, u], out_hbm.at[half_of(0, u)], ssems[u])

    def t_body(t, carry):
        g = 3 * t
        do_chunk(g, 0)
        do_chunk(g + 1, 1)
        do_chunk(g + 2, 2)
        return carry

    lax.fori_loop(0, _NCHUNK // 3 - 1, t_body, 0, unroll=False)

    do_chunk(_NCHUNK - 3, 0)
    do_chunk(_NCHUNK - 2, 1, tail=True)
    do_chunk(_NCHUNK - 1, 2, tail=True)
    for u in range(2):
        wait_scatter(_NCHUNK - 1, u)


def kernel(x, row_embed, col_embed):
    b, c, h, w = x.shape
    emb = jnp.concatenate([row_embed.T, col_embed.T], axis=0)
    x2 = x.reshape(b * c * h, w)

    mesh = plsc.VectorSubcoreMesh(core_axis_name="c", subcore_axis_name="s")
    run = pl.kernel(
        _sc_body,
        mesh=mesh,
        out_type=jax.ShapeDtypeStruct((b * c * h, w), jnp.float32),
        scratch_types=[
            pltpu.VMEM((_PPW, _W), jnp.float32),
            pltpu.VMEM((_HH, _W), jnp.float32),
            pltpu.VMEM((_HH, _W), jnp.float32),
            pltpu.VMEM((_HH, _W), jnp.float32),
            pltpu.VMEM_SHARED((_NS, 2, _HQ, _W), jnp.float32),
            pltpu.SemaphoreType.DMA,
            pltpu.SemaphoreType.DMA,
            pltpu.SemaphoreType.DMA,
            pltpu.SemaphoreType.DMA,
            pltpu.SemaphoreType.DMA,
        ],
    )
    out2 = run(x2, emb)
    return out2.reshape(b, c, h, w)
